# baseline (speedup 1.0000x reference)
import functools

import jax
import jax.numpy as jnp
from jax import lax
from jax.experimental import pallas as pl
from jax.experimental.pallas import tpu as pltpu
from jax.experimental.pallas import tpu_sc as plsc

_N = 10000
_D = 256
_E = 80000
_E2 = 2 * _E
_CHUNK = 64
_NSUB = 16
_NCORE = 2
_NCHUNK = 160
_EPT = _CHUNK * _NCHUNK
_E2P = _EPT * _NSUB
_RPT = 632
_NPAD = _RPT * _NSUB
_BN = 1000


def _make_sc_scatter(width):
    def body(pidx_hbm, tab_hbm, zsrc_hbm, out_hbm,
             inc_sh, pidx_v, g0, g1, g2, g3, d0, d1, d2, d3,
             r0, r1, r2, r3,
             gs0, gs1, gs2, gs3, ss0, ss1, ss2, ss3):
        c = lax.axis_index("c")
        s = lax.axis_index("s")
        gbufs = (g0, g1, g2, g3)
        dbufs = (d0, d1, d2, d3)
        rbufs = (r0, r1, r2, r3)
        gsems = (gs0, gs1, gs2, gs3)
        ssems = (ss0, ss1, ss2, ss3)
        dummy = tab_hbm.at[pl.ds(0, _CHUNK)]

        pltpu.sync_copy(pidx_hbm.at[c, s], pidx_v)
        pltpu.sync_copy(zsrc_hbm, inc_sh.at[pl.ds(s * _RPT, _RPT)])
        plsc.subcore_barrier()

        def unpack(row, half, slot):
            for k in range(_CHUNK // 16):
                v = pidx_v[row, pl.ds(64 * half + 16 * k, 16)]
                gbufs[slot][pl.ds(16 * k, 16)] = v & 0xFFFF
                dbufs[slot][pl.ds(16 * k, 16)] = lax.shift_right_logical(v, 16)

        unpack(0, 0, 0)
        unpack(0, 1, 1)
        unpack(1, 0, 2)
        pltpu.async_copy(tab_hbm.at[g0], r0, gs0)
        pltpu.async_copy(tab_hbm.at[g1], r1, gs1)
        pltpu.async_copy(tab_hbm.at[g2], r2, gs2)

        def group(g, carry):
            for k in range(4):
                i = 4 * g + k
                c3 = (k + 3) % 4

                pltpu.make_async_copy(dummy, rbufs[k], gsems[k]).wait()
                pltpu.async_copy(rbufs[k], inc_sh.at[dbufs[k]], ssems[k],
                                 add=True)

                @pl.when(i >= 1)
                def _():
                    pltpu.make_async_copy(dummy, rbufs[c3], ssems[c3]).wait()

                @pl.when(i + 3 < _NCHUNK)
                def _():
                    unpack(2 * g + (k + 3) // 2, (k + 3) % 2, c3)
                    pltpu.async_copy(tab_hbm.at[gbufs[c3]], rbufs[c3],
                                     gsems[c3])
            return carry

        lax.fori_loop(0, _NCHUNK // 4, group, 0)
        pltpu.make_async_copy(dummy, rbufs[(_NCHUNK - 1) % 4],
                              ssems[(_NCHUNK - 1) % 4]).wait()
        plsc.subcore_barrier()
        pltpu.sync_copy(inc_sh.at[pl.ds(s * _RPT, _RPT)],
                        out_hbm.at[c, pl.ds(s * _RPT, _RPT)])

    return pl.kernel(
        body,
        mesh=plsc.VectorSubcoreMesh(core_axis_name="c", subcore_axis_name="s"),
        out_type=jax.ShapeDtypeStruct((_NCORE, _NPAD, width), jnp.float32),
        scratch_types=[
            pltpu.VMEM_SHARED((_NPAD, width), jnp.float32),
            pltpu.VMEM((_NCHUNK // 2, 2 * _CHUNK), jnp.int32),
            pltpu.VMEM((_CHUNK,), jnp.int32),
            pltpu.VMEM((_CHUNK,), jnp.int32),
            pltpu.VMEM((_CHUNK,), jnp.int32),
            pltpu.VMEM((_CHUNK,), jnp.int32),
            pltpu.VMEM((_CHUNK,), jnp.int32),
            pltpu.VMEM((_CHUNK,), jnp.int32),
            pltpu.VMEM((_CHUNK,), jnp.int32),
            pltpu.VMEM((_CHUNK,), jnp.int32),
            pltpu.VMEM((_CHUNK, width), jnp.float32),
            pltpu.VMEM((_CHUNK, width), jnp.float32),
            pltpu.VMEM((_CHUNK, width), jnp.float32),
            pltpu.VMEM((_CHUNK, width), jnp.float32),
            pltpu.SemaphoreType.DMA,
            pltpu.SemaphoreType.DMA,
            pltpu.SemaphoreType.DMA,
            pltpu.SemaphoreType.DMA,
            pltpu.SemaphoreType.DMA,
            pltpu.SemaphoreType.DMA,
            pltpu.SemaphoreType.DMA,
            pltpu.SemaphoreType.DMA,
        ],
    )


_sc_scatter = _make_sc_scatter(128)


def _mm_body(x_ref, w_ref, b_ref, o_ref):
    o_ref[0] = (jnp.dot(x_ref[...], w_ref[0], preferred_element_type=jnp.float32)
                + b_ref[0])


def _msg_mm(h, wc, bc):
    return pl.pallas_call(
        _mm_body,
        grid=(4, _N // _BN),
        in_specs=[
            pl.BlockSpec((_BN, _D), lambda q, i: (i, 0)),
            pl.BlockSpec((1, _D, 128), lambda q, i: (q, 0, 0)),
            pl.BlockSpec((1, 1, 128), lambda q, i: (q, 0, 0)),
        ],
        out_specs=pl.BlockSpec((1, _BN, 128), lambda q, i: (q, i, 0)),
        out_shape=jax.ShapeDtypeStruct((4, _N, 128), jnp.float32),
    )(h, wc, bc)


def _gru_body_res(inc_ref, cnt_ref, h_ref, res_ref, wih_ref, whh_ref,
                  bih_ref, bhh_ref, o_ref):
    inv = 1.0 / (cnt_ref[...] + 1e-7)
    inc = jnp.concatenate([inc_ref[0], inc_ref[1]], axis=1) * inv
    x = jnp.concatenate([res_ref[...], inc], axis=1)
    _gru_core(x, h_ref, wih_ref, whh_ref, bih_ref, bhh_ref, o_ref)


def _gru_body_nores(inc_ref, cnt_ref, h_ref, wih_ref, whh_ref,
                    bih_ref, bhh_ref, o_ref):
    inv = 1.0 / (cnt_ref[...] + 1e-7)
    x = jnp.concatenate([inc_ref[0], inc_ref[1]], axis=1) * inv
    _gru_core(x, h_ref, wih_ref, whh_ref, bih_ref, bhh_ref, o_ref)


def _gru_core(x, h_ref, wih_ref, whh_ref, bih_ref, bhh_ref, o_ref):
    h = h_ref[...]
    gi = jnp.dot(x, wih_ref[...], preferred_element_type=jnp.float32) + bih_ref[0]
    gh = jnp.dot(h, whh_ref[...], preferred_element_type=jnp.float32) + bhh_ref[0]
    r = jax.nn.sigmoid(gi[:, :_D] + gh[:, :_D])
    z = jax.nn.sigmoid(gi[:, _D:2 * _D] + gh[:, _D:2 * _D])
    n = jnp.tanh(gi[:, 2 * _D:] + r * gh[:, 2 * _D:])
    o_ref[...] = (1.0 - z) * n + z * h


def _gru(inc, cnt, h, res, wihT, whhT, bih, bhh):
    din = wihT.shape[0]
    specs = [
        pl.BlockSpec((_NCORE, _BN, 128), lambda i: (0, i, 0)),
        pl.BlockSpec((_BN, 1), lambda i: (i, 0)),
        pl.BlockSpec((_BN, _D), lambda i: (i, 0)),
    ]
    args = [inc, cnt, h]
    if res is not None:
        specs.append(pl.BlockSpec((_BN, _D), lambda i: (i, 0)))
        args.append(res)
    specs += [
        pl.BlockSpec((din, 3 * _D), lambda i: (0, 0)),
        pl.BlockSpec((_D, 3 * _D), lambda i: (0, 0)),
        pl.BlockSpec((1, 3 * _D), lambda i: (0, 0)),
        pl.BlockSpec((1, 3 * _D), lambda i: (0, 0)),
    ]
    args += [wihT, whhT, bih, bhh]
    body = _gru_body_res if res is not None else _gru_body_nores
    return pl.pallas_call(
        body,
        grid=(_N // _BN,),
        in_specs=specs,
        out_specs=pl.BlockSpec((_BN, _D), lambda i: (i, 0)),
        out_shape=jax.ShapeDtypeStruct((_N, _D), jnp.float32),
    )(*args)


def _build_wcat(w0, w1, b0, b1):
    wt0, wt1 = w0.T, w1.T
    wc = jnp.stack([wt0[:, :128], wt1[:, :128], wt0[:, 128:], wt1[:, 128:]])
    bc = jnp.stack([b0[:128], b1[:128], b0[128:], b1[128:]]).reshape(4, 1, 128)
    return wc, bc


def kernel(initial_node_representation, adj0, adj1, Wm00, bm00, Wm01, bm01,
           Wm10, bm10, Wm11, bm11, W_ih0, W_hh0, b_ih0, b_hh0,
           W_ih1, W_hh1, b_ih1, b_hh1):
    x = initial_node_representation
    a0 = adj0.astype(jnp.int32)
    a1 = adj1.astype(jnp.int32)
    src = jnp.concatenate([a0[:, 0], a1[:, 0] + _N])
    dst = jnp.concatenate([a0[:, 1], a1[:, 1]])
    pad = _E2P - _E2
    srcp = jnp.concatenate([src, jnp.zeros((pad,), jnp.int32)])
    dstp = jnp.concatenate([dst, jnp.full((pad,), _N, jnp.int32)])
    gidx2 = jnp.stack([srcp, srcp + 2 * _N])
    pidx = (gidx2 | (dstp << 16)[None]).reshape(
        _NCORE, _NSUB, _NCHUNK // 2, 2 * _CHUNK)
    ones_tab = jnp.ones((4 * _N, 128), jnp.float32)
    zsrc = jnp.zeros((_RPT, 128), jnp.float32)

    cnt_out = _sc_scatter(pidx, ones_tab, zsrc)
    cnt = cnt_out[0, :, 0:1]

    wc0, bc0 = _build_wcat(Wm00, Wm01, bm00, bm01)
    wc1, bc1 = _build_wcat(Wm10, Wm11, bm10, bm11)
    wihT0, whhT0 = W_ih0.T, W_hh0.T
    wihT1, whhT1 = W_ih1.T, W_hh1.T
    bih0, bhh0 = b_ih0.reshape(1, -1), b_hh0.reshape(1, -1)
    bih1, bhh1 = b_ih1.reshape(1, -1), b_hh1.reshape(1, -1)

    h = x
    for _ in range(3):
        m = _msg_mm(h, wc0, bc0)
        inc = _sc_scatter(pidx, m.reshape(4 * _N, 128), zsrc)
        h = _gru(inc, cnt, h, None, wihT0, whhT0, bih0, bhh0)
    for _ in range(3):
        m = _msg_mm(h, wc1, bc1)
        inc = _sc_scatter(pidx, m.reshape(4 * _N, 128), zsrc)
        h = _gru(inc, cnt, h, x, wihT1, whhT1, bih1, bhh1)
    return h


# speedup vs baseline: 1.6637x; 1.6637x over previous
import functools

import jax
import jax.numpy as jnp
from jax import lax
from jax.experimental import pallas as pl
from jax.experimental.pallas import tpu as pltpu
from jax.experimental.pallas import tpu_sc as plsc

_N = 10000
_D = 256
_E = 80000
_E2 = 2 * _E
_CHUNK = 80
_NSUB = 16
_NCORE = 2
_NCHUNK = 126
_EPT = _CHUNK * _NCHUNK
_E2P = _EPT * _NSUB
_RPT = 632
_NPAD = _RPT * _NSUB
_BN = 1000


def _make_sc_scatter(width):
    def body(pidx_hbm, tab_hbm, zsrc_hbm, out_hbm,
             inc_sh, pidx_v, g0, g1, g2, d0, d1, d2, r0, r1, r2,
             gs0, gs1, gs2, ss0, ss1, ss2):
        c = lax.axis_index("c")
        s = lax.axis_index("s")
        gbufs = (g0, g1, g2)
        dbufs = (d0, d1, d2)
        rbufs = (r0, r1, r2)
        gsems = (gs0, gs1, gs2)
        ssems = (ss0, ss1, ss2)
        dummy = tab_hbm.at[pl.ds(0, _CHUNK)]

        pltpu.sync_copy(pidx_hbm.at[c, s], pidx_v)
        pltpu.sync_copy(zsrc_hbm, inc_sh.at[pl.ds(s * _RPT, _RPT)])
        plsc.subcore_barrier()

        def unpack(i, slot):
            for k in range(_CHUNK // 16):
                v = pidx_v[i, pl.ds(16 * k, 16)]
                gbufs[slot][pl.ds(16 * k, 16)] = v & 0xFFFF
                dbufs[slot][pl.ds(16 * k, 16)] = lax.shift_right_logical(v, 16)

        unpack(0, 0)
        unpack(1, 1)
        pltpu.async_copy(tab_hbm.at[g0], r0, gs0)
        pltpu.async_copy(tab_hbm.at[g1], r1, gs1)

        def group(g, carry):
            for k in range(3):
                i = 3 * g + k
                c2 = (k + 2) % 3

                pltpu.make_async_copy(dummy, rbufs[k], gsems[k]).wait()
                pltpu.async_copy(rbufs[k], inc_sh.at[dbufs[k]], ssems[k],
                                 add=True)

                @pl.when(i >= 1)
                def _():
                    pltpu.make_async_copy(dummy, rbufs[c2], ssems[c2]).wait()

                @pl.when(i + 2 < _NCHUNK)
                def _():
                    unpack(i + 2, c2)
                    pltpu.async_copy(tab_hbm.at[gbufs[c2]], rbufs[c2],
                                     gsems[c2])
            return carry

        lax.fori_loop(0, _NCHUNK // 3, group, 0)
        pltpu.make_async_copy(dummy, rbufs[(_NCHUNK - 1) % 3],
                              ssems[(_NCHUNK - 1) % 3]).wait()
        plsc.subcore_barrier()
        pltpu.sync_copy(inc_sh.at[pl.ds(s * _RPT, _RPT)],
                        out_hbm.at[c, pl.ds(s * _RPT, _RPT)])

    return pl.kernel(
        body,
        mesh=plsc.VectorSubcoreMesh(core_axis_name="c", subcore_axis_name="s"),
        out_type=jax.ShapeDtypeStruct((_NCORE, _NPAD, width), jnp.float32),
        scratch_types=[
            pltpu.VMEM_SHARED((_NPAD, width), jnp.float32),
            pltpu.VMEM((_NCHUNK, _CHUNK), jnp.int32),
            pltpu.VMEM((_CHUNK,), jnp.int32),
            pltpu.VMEM((_CHUNK,), jnp.int32),
            pltpu.VMEM((_CHUNK,), jnp.int32),
            pltpu.VMEM((_CHUNK,), jnp.int32),
            pltpu.VMEM((_CHUNK,), jnp.int32),
            pltpu.VMEM((_CHUNK,), jnp.int32),
            pltpu.VMEM((_CHUNK, width), jnp.float32),
            pltpu.VMEM((_CHUNK, width), jnp.float32),
            pltpu.VMEM((_CHUNK, width), jnp.float32),
            pltpu.SemaphoreType.DMA,
            pltpu.SemaphoreType.DMA,
            pltpu.SemaphoreType.DMA,
            pltpu.SemaphoreType.DMA,
            pltpu.SemaphoreType.DMA,
            pltpu.SemaphoreType.DMA,
        ],
    )


_sc_scatter = _make_sc_scatter(128)


def _mm_body(x_ref, w_ref, b_ref, o_ref):
    o_ref[0] = (jnp.dot(x_ref[...], w_ref[0], preferred_element_type=jnp.float32)
                + b_ref[0])


def _msg_mm(h, wc, bc):
    return pl.pallas_call(
        _mm_body,
        grid=(4, _N // _BN),
        in_specs=[
            pl.BlockSpec((_BN, _D), lambda q, i: (i, 0)),
            pl.BlockSpec((1, _D, 128), lambda q, i: (q, 0, 0)),
            pl.BlockSpec((1, 1, 128), lambda q, i: (q, 0, 0)),
        ],
        out_specs=pl.BlockSpec((1, _BN, 128), lambda q, i: (q, i, 0)),
        out_shape=jax.ShapeDtypeStruct((4, _N, 128), jnp.float32),
    )(h, wc, bc)


def _gru_core(inc_ref, cnt_ref, h_ref, res_ref, wih_ref, whh_ref,
              bih_ref, bhh_ref):
    inv = 1.0 / (cnt_ref[...] + 1e-7)
    inc = jnp.concatenate([inc_ref[0], inc_ref[1]], axis=1) * inv
    x = inc if res_ref is None else jnp.concatenate([res_ref[...], inc], axis=1)
    h = h_ref[...]
    gi = jnp.dot(x, wih_ref[...], preferred_element_type=jnp.float32) + bih_ref[0]
    gh = jnp.dot(h, whh_ref[...], preferred_element_type=jnp.float32) + bhh_ref[0]
    r = jax.nn.sigmoid(gi[:, :_D] + gh[:, :_D])
    z = jax.nn.sigmoid(gi[:, _D:2 * _D] + gh[:, _D:2 * _D])
    n = jnp.tanh(gi[:, 2 * _D:] + r * gh[:, 2 * _D:])
    return (1.0 - z) * n + z * h


def _gru_specs_args(inc, cnt, h, res, wihT, whhT, bih, bhh):
    din = wihT.shape[0]
    specs = [
        pl.BlockSpec((_NCORE, _BN, 128), lambda i: (0, i, 0)),
        pl.BlockSpec((_BN, 1), lambda i: (i, 0)),
        pl.BlockSpec((_BN, _D), lambda i: (i, 0)),
    ]
    args = [inc, cnt, h]
    if res is not None:
        specs.append(pl.BlockSpec((_BN, _D), lambda i: (i, 0)))
        args.append(res)
    specs += [
        pl.BlockSpec((din, 3 * _D), lambda i: (0, 0)),
        pl.BlockSpec((_D, 3 * _D), lambda i: (0, 0)),
        pl.BlockSpec((1, 3 * _D), lambda i: (0, 0)),
        pl.BlockSpec((1, 3 * _D), lambda i: (0, 0)),
    ]
    args += [wihT, whhT, bih, bhh]
    return specs, args


def _gru(inc, cnt, h, res, wihT, whhT, bih, bhh):
    specs, args = _gru_specs_args(inc, cnt, h, res, wihT, whhT, bih, bhh)

    if res is not None:
        def body(inc_ref, cnt_ref, h_ref, res_ref, wih_ref, whh_ref,
                 bih_ref, bhh_ref, o_ref):
            o_ref[...] = _gru_core(inc_ref, cnt_ref, h_ref, res_ref,
                                   wih_ref, whh_ref, bih_ref, bhh_ref)
    else:
        def body(inc_ref, cnt_ref, h_ref, wih_ref, whh_ref,
                 bih_ref, bhh_ref, o_ref):
            o_ref[...] = _gru_core(inc_ref, cnt_ref, h_ref, None,
                                   wih_ref, whh_ref, bih_ref, bhh_ref)

    return pl.pallas_call(
        body,
        grid=(_N // _BN,),
        in_specs=specs,
        out_specs=pl.BlockSpec((_BN, _D), lambda i: (i, 0)),
        out_shape=jax.ShapeDtypeStruct((_N, _D), jnp.float32),
    )(*args)


def _gru_mm(inc, cnt, h, res, wihT, whhT, bih, bhh, wcn, bcn):
    specs, args = _gru_specs_args(inc, cnt, h, res, wihT, whhT, bih, bhh)
    specs += [
        pl.BlockSpec((4, _D, 128), lambda i: (0, 0, 0)),
        pl.BlockSpec((4, 1, 128), lambda i: (0, 0, 0)),
    ]
    args += [wcn, bcn]

    def emit(hn, wcn_ref, bcn_ref, oh_ref, om_ref):
        oh_ref[...] = hn
        for q in range(4):
            om_ref[q] = (jnp.dot(hn, wcn_ref[q],
                                 preferred_element_type=jnp.float32)
                         + bcn_ref[q])

    if res is not None:
        def body(inc_ref, cnt_ref, h_ref, res_ref, wih_ref, whh_ref,
                 bih_ref, bhh_ref, wcn_ref, bcn_ref, oh_ref, om_ref):
            hn = _gru_core(inc_ref, cnt_ref, h_ref, res_ref,
                           wih_ref, whh_ref, bih_ref, bhh_ref)
            emit(hn, wcn_ref, bcn_ref, oh_ref, om_ref)
    else:
        def body(inc_ref, cnt_ref, h_ref, wih_ref, whh_ref,
                 bih_ref, bhh_ref, wcn_ref, bcn_ref, oh_ref, om_ref):
            hn = _gru_core(inc_ref, cnt_ref, h_ref, None,
                           wih_ref, whh_ref, bih_ref, bhh_ref)
            emit(hn, wcn_ref, bcn_ref, oh_ref, om_ref)

    return pl.pallas_call(
        body,
        grid=(_N // _BN,),
        in_specs=specs,
        out_specs=[
            pl.BlockSpec((_BN, _D), lambda i: (i, 0)),
            pl.BlockSpec((4, _BN, 128), lambda i: (0, i, 0)),
        ],
        out_shape=[
            jax.ShapeDtypeStruct((_N, _D), jnp.float32),
            jax.ShapeDtypeStruct((4, _N, 128), jnp.float32),
        ],
    )(*args)


def _build_wcat(w0, w1, b0, b1):
    wt0, wt1 = w0.T, w1.T
    wc = jnp.stack([wt0[:, :128], wt1[:, :128], wt0[:, 128:], wt1[:, 128:]])
    bc = jnp.stack([b0[:128], b1[:128], b0[128:], b1[128:]]).reshape(4, 1, 128)
    return wc, bc


def kernel(initial_node_representation, adj0, adj1, Wm00, bm00, Wm01, bm01,
           Wm10, bm10, Wm11, bm11, W_ih0, W_hh0, b_ih0, b_hh0,
           W_ih1, W_hh1, b_ih1, b_hh1):
    x = initial_node_representation
    a0 = adj0.astype(jnp.int32)
    a1 = adj1.astype(jnp.int32)
    src = jnp.concatenate([a0[:, 0], a1[:, 0] + _N])
    dst = jnp.concatenate([a0[:, 1], a1[:, 1]])
    pad = _E2P - _E2
    srcp = jnp.concatenate([src, jnp.zeros((pad,), jnp.int32)])
    dstp = jnp.concatenate([dst, jnp.full((pad,), _N, jnp.int32)])
    gidx2 = jnp.stack([srcp, srcp + 2 * _N])
    pidx = (gidx2 | (dstp << 16)[None]).reshape(_NCORE, _NSUB, _NCHUNK, _CHUNK)
    ones_tab = jnp.ones((4 * _N, 128), jnp.float32)
    zsrc = jnp.zeros((_RPT, 128), jnp.float32)

    cnt_out = _sc_scatter(pidx, ones_tab, zsrc)
    cnt = cnt_out[0, :, 0:1]

    wc0, bc0 = _build_wcat(Wm00, Wm01, bm00, bm01)
    wc1, bc1 = _build_wcat(Wm10, Wm11, bm10, bm11)
    wihT0, whhT0 = W_ih0.T, W_hh0.T
    wihT1, whhT1 = W_ih1.T, W_hh1.T
    bih0, bhh0 = b_ih0.reshape(1, -1), b_hh0.reshape(1, -1)
    bih1, bhh1 = b_ih1.reshape(1, -1), b_hh1.reshape(1, -1)

    h = x
    m = _msg_mm(h, wc0, bc0)
    for t in range(6):
        inc = _sc_scatter(pidx, m.reshape(4 * _N, 128), zsrc)
        res = x if t >= 3 else None
        gw = ((wihT1, whhT1, bih1, bhh1) if t >= 3
              else (wihT0, whhT0, bih0, bhh0))
        if t == 5:
            h = _gru(inc, cnt, h, res, *gw)
        else:
            wcn, bcn = (wc1, bc1) if t + 1 >= 3 else (wc0, bc0)
            h, m = _gru_mm(inc, cnt, h, res, *gw, wcn, bcn)
    return h
